# baseline (speedup 1.0000x reference)
import functools

import jax
import jax.numpy as jnp
from jax import lax
from jax.experimental import pallas as pl
from jax.experimental.pallas import tpu as pltpu
from jax.experimental.pallas import tpu_sc as plsc

N = 10000
E = 320000
D = 128
CH = 80
NC = 2
NS = 16
NW = NC * NS
CPW = 128
EP = NW * CPW * CH
NP = 10240
RPT = NP // NS

_f32 = jnp.float32
_i32 = jnp.int32


def _mesh():
    return plsc.VectorSubcoreMesh(
        core_axis_name="c", subcore_axis_name="s",
        num_cores=NC, num_subcores=NS)


def _sc_deg_body(dst_hbm, ew_hbm, out_hbm, dst_v, ew_v, bc_v, deg_sh):
    c = lax.axis_index("c")
    s = lax.axis_index("s")
    wid = s * NC + c

    def zrow(r, carry):
        bc_v[r, :] = jnp.zeros((16,), _f32)
        return carry

    lax.fori_loop(0, CH, zrow, 0)
    for t in range(RPT // CH):
        pltpu.sync_copy(bc_v, deg_sh.at[pl.ds(s * RPT + t * CH, CH)])
    plsc.subcore_barrier()

    def chunk(i, carry):
        base = (wid + NW * i) * CH
        pltpu.sync_copy(dst_hbm.at[pl.ds(base, CH)], dst_v)
        pltpu.sync_copy(ew_hbm.at[pl.ds(base, CH)], ew_v)

        def grp(g, cc):
            ew16 = ew_v[pl.ds(g * 16, 16)]
            for l in range(16):
                bc_v[g * 16 + l, :] = jnp.full((16,), ew16[l], _f32)
            return cc

        lax.fori_loop(0, CH // 16, grp, 0)
        pltpu.sync_copy(bc_v, deg_sh.at[dst_v], add=True)
        return carry

    lax.fori_loop(0, CPW, chunk, 0)
    plsc.subcore_barrier()
    for t in range(RPT // CH):
        r0 = s * RPT + t * CH
        pltpu.sync_copy(deg_sh.at[pl.ds(r0, CH)],
                        out_hbm.at[c, pl.ds(r0, CH)])


def _deg_call(dst, ew):
    return pl.kernel(
        _sc_deg_body,
        out_type=jax.ShapeDtypeStruct((NC, NP, 16), _f32),
        mesh=_mesh(),
        compiler_params=pltpu.CompilerParams(use_tc_tiling_on_sc=False),
        scratch_types=[
            pltpu.VMEM((CH,), _i32),
            pltpu.VMEM((CH,), _f32),
            pltpu.VMEM((CH, 16), _f32),
            pltpu.VMEM_SHARED((NP, 16), _f32),
        ],
    )(dst, ew)


def _sc_scatter_body(hp_hbm, src_hbm, dst_hbm, ew_hbm, out_hbm, *scr):
    bufs = [tuple(scr[q * 4:q * 4 + 4]) for q in range(4)]
    agg_sh = scr[16]
    gsem = scr[17:21]
    ssem = scr[21:25]
    isem = scr[25]
    c = lax.axis_index("c")
    s = lax.axis_index("s")
    wid = s * NC + c
    rows0 = bufs[0][3]

    def zrow(r, carry):
        for j in range(8):
            rows0[r, pl.ds(j * 16, 16)] = jnp.zeros((16,), _f32)
        return carry

    lax.fori_loop(0, CH, zrow, 0)
    for t in range(RPT // CH):
        pltpu.sync_copy(rows0, agg_sh.at[pl.ds(s * RPT + t * CH, CH)])
    plsc.subcore_barrier()

    def issue_idx(i, buf):
        sv, dv, ev, _ = buf
        b = (wid + NW * i) * CH
        pltpu.async_copy(src_hbm.at[pl.ds(b, CH)], sv, isem)
        pltpu.async_copy(dst_hbm.at[pl.ds(b, CH)], dv, isem)
        pltpu.async_copy(ew_hbm.at[pl.ds(b, CH)], ev, isem)

    def wait_idx(buf):
        sv, dv, ev, _ = buf
        pltpu.make_async_copy(src_hbm.at[pl.ds(0, CH)], sv, isem).wait()
        pltpu.make_async_copy(dst_hbm.at[pl.ds(0, CH)], dv, isem).wait()
        pltpu.make_async_copy(ew_hbm.at[pl.ds(0, CH)], ev, isem).wait()

    def scale(buf):
        _, _, ev, rows = buf

        def grp(g, cc):
            ew16 = ev[pl.ds(g * 16, 16)]
            for l in range(16):
                e = g * 16 + l
                sp = ew16[l]
                for j in range(8):
                    sl = pl.ds(j * 16, 16)
                    rows[e, sl] = rows[e, sl] * sp
            return cc

        lax.fori_loop(0, CH // 16, grp, 0)

    def issue_gather(buf, sem):
        pltpu.async_copy(hp_hbm.at[buf[0]], buf[3], sem)

    def wait_gather(buf, sem):
        pltpu.make_async_copy(hp_hbm.at[buf[0]], buf[3], sem).wait()

    def issue_scatter(buf, sem):
        pltpu.async_copy(buf[3], agg_sh.at[buf[1]], sem, add=True)

    def wait_scatter(buf, sem):
        pltpu.make_async_copy(buf[3], agg_sh.at[buf[1]], sem).wait()

    issue_idx(0, bufs[0])
    issue_idx(1, bufs[1])
    wait_idx(bufs[0])
    issue_gather(bufs[0], gsem[0])
    wait_idx(bufs[1])
    issue_gather(bufs[1], gsem[1])

    def half(i, q, drain, issue):
        p2 = (q + 2) % 4
        cur = bufs[q]
        nx2 = bufs[p2]
        wait_gather(cur, gsem[q])
        if drain:
            wait_scatter(nx2, ssem[p2])
        if issue:
            issue_idx(i + 2, nx2)
        scale(cur)
        if issue:
            wait_idx(nx2)
            issue_gather(nx2, gsem[p2])
        issue_scatter(cur, ssem[q])

    for q in range(4):
        half(q, q, q >= 2, True)

    def quad(k, carry):
        for q in range(4):
            half(4 * k + q, q, True, True)
        return carry

    lax.fori_loop(1, CPW // 4 - 1, quad, 0)
    for q in range(4):
        half(CPW - 4 + q, q, True, q < 2)
    wait_scatter(bufs[2], ssem[2])
    wait_scatter(bufs[3], ssem[3])

    plsc.subcore_barrier()
    for t in range(RPT // CH):
        r0 = s * RPT + t * CH
        pltpu.sync_copy(agg_sh.at[pl.ds(r0, CH)],
                        out_hbm.at[c, pl.ds(r0, CH)])


def _scatter_call(hp, src, dst, ew):
    slot = [pltpu.VMEM((CH,), _i32), pltpu.VMEM((CH,), _i32),
            pltpu.VMEM((CH,), _f32), pltpu.VMEM((CH, D), _f32)]
    return pl.kernel(
        _sc_scatter_body,
        out_type=jax.ShapeDtypeStruct((NC, NP, D), _f32),
        mesh=_mesh(),
        scratch_types=(slot * 4
                       + [pltpu.VMEM_SHARED((NP, D), _f32)]
                       + [pltpu.SemaphoreType.DMA] * 9),
    )(hp, src, dst, ew)


def _tc_pre_body(degp_ref, x_ref, w_ref, b_ref, hp_ref, dis_ref):
    a = degp_ref[...]
    deg = a[0, :N, 0:1] + a[1, :N, 0:1]
    dis = jnp.where(deg > 0, lax.rsqrt(deg), 0.0)
    h = jnp.dot(x_ref[...], w_ref[...], preferred_element_type=_f32)
    hp_ref[...] = (h + b_ref[...]) * dis
    dis_ref[...] = dis


def _tc_pre_call(degp, x, w, b):
    return pl.pallas_call(
        _tc_pre_body,
        out_shape=[jax.ShapeDtypeStruct((N, D), _f32),
                   jax.ShapeDtypeStruct((N, 1), _f32)],
    )(degp, x, w, b)


def _tc_mid_body(aggp_ref, dis_ref, g_ref, be_ref, w_ref, b_ref, hp_ref):
    a = aggp_ref[...]
    dis = dis_ref[...]
    out = (a[0, :N] + a[1, :N]) * dis
    mu = jnp.mean(out, axis=0, keepdims=True)
    xc = out - mu
    var = jnp.mean(xc * xc, axis=0, keepdims=True)
    y = xc * (g_ref[...] / jnp.sqrt(var + 1e-5)) + be_ref[...]
    y = jnp.maximum(y, 0.0)
    h = jnp.dot(y, w_ref[...], preferred_element_type=_f32)
    hp_ref[...] = (h + b_ref[...]) * dis


def _tc_mid_call(aggp, dis, g, be, w, b):
    return pl.pallas_call(
        _tc_mid_body,
        out_shape=jax.ShapeDtypeStruct((N, D), _f32),
    )(aggp, dis, g, be, w, b)


def _tc_post_body(aggp_ref, dis_ref, g_ref, be_ref, y_ref):
    a = aggp_ref[...]
    out = (a[0, :N] + a[1, :N]) * dis_ref[...]
    mu = jnp.mean(out, axis=0, keepdims=True)
    xc = out - mu
    var = jnp.mean(xc * xc, axis=0, keepdims=True)
    y_ref[...] = xc * (g_ref[...] / jnp.sqrt(var + 1e-5)) + be_ref[...]


def _tc_post_call(aggp, dis, g, be):
    return pl.pallas_call(
        _tc_post_body,
        out_shape=jax.ShapeDtypeStruct((N, D), _f32),
    )(aggp, dis, g, be)


def kernel(x, edge_index, edge_attr, W0, b0, gamma0, beta0, W1, b1, gamma1, beta1):
    pad = EP - E
    src = jnp.concatenate([edge_index[0].astype(_i32),
                           jnp.zeros((pad,), _i32)])
    dst = jnp.concatenate([edge_index[1].astype(_i32),
                           jnp.zeros((pad,), _i32)])
    ew = jnp.concatenate([edge_attr[:, 0], jnp.zeros((pad,), _f32)])

    degp = _deg_call(dst, ew)
    hp0, dis = _tc_pre_call(degp, x, W0, b0.reshape(1, D))
    agg0 = _scatter_call(hp0, src, dst, ew)
    hp1 = _tc_mid_call(agg0, dis, gamma0.reshape(1, D), beta0.reshape(1, D),
                       W1, b1.reshape(1, D))
    agg1 = _scatter_call(hp1, src, dst, ew)
    return _tc_post_call(agg1, dis, gamma1.reshape(1, D), beta1.reshape(1, D))


# speedup vs baseline: 1.0102x; 1.0102x over previous
import functools

import jax
import jax.numpy as jnp
from jax import lax
from jax.experimental import pallas as pl
from jax.experimental.pallas import tpu as pltpu
from jax.experimental.pallas import tpu_sc as plsc

N = 10000
E = 320000
D = 128
NC = 2
NS = 16
NW = NC * NS
CH = 128
CPW = 80
EP = NW * CPW * CH
NCHUNK = EP // CH
CHD = 400
CPWD = E // (NW * CHD)
NP = 10240
RPT = NP // NS

_f32 = jnp.float32
_i32 = jnp.int32


def _mesh():
    return plsc.VectorSubcoreMesh(
        core_axis_name="c", subcore_axis_name="s",
        num_cores=NC, num_subcores=NS)


_sc_params = pltpu.CompilerParams(use_tc_tiling_on_sc=False)


def _sc_deg_body(dst_hbm, ew_hbm, out_hbm, dst_v, ew_v, bc_v, deg_sh):
    c = lax.axis_index("c")
    s = lax.axis_index("s")
    wid = s * NC + c

    def zrow(r, carry):
        bc_v[r, :] = jnp.zeros((16,), _f32)
        return carry

    lax.fori_loop(0, CHD, zrow, 0)
    pltpu.sync_copy(bc_v, deg_sh.at[pl.ds(s * RPT, CHD)])
    pltpu.sync_copy(bc_v.at[pl.ds(0, RPT - CHD)],
                    deg_sh.at[pl.ds(s * RPT + CHD, RPT - CHD)])
    plsc.subcore_barrier()

    def chunk(i, carry):
        base = (wid + NW * i) * CHD
        pltpu.sync_copy(dst_hbm.at[pl.ds(base, CHD)], dst_v)
        pltpu.sync_copy(ew_hbm.at[pl.ds(base, CHD)], ew_v)

        def grp(g, cc):
            ew16 = ew_v[pl.ds(g * 16, 16)]
            for l in range(16):
                bc_v[g * 16 + l, :] = jnp.full((16,), ew16[l], _f32)
            return cc

        lax.fori_loop(0, CHD // 16, grp, 0)
        pltpu.sync_copy(bc_v, deg_sh.at[dst_v], add=True)
        return carry

    lax.fori_loop(0, CPWD, chunk, 0)
    plsc.subcore_barrier()
    pltpu.sync_copy(deg_sh.at[pl.ds(s * RPT, CHD)],
                    out_hbm.at[c, pl.ds(s * RPT, CHD)])
    pltpu.sync_copy(deg_sh.at[pl.ds(s * RPT + CHD, RPT - CHD)],
                    out_hbm.at[c, pl.ds(s * RPT + CHD, RPT - CHD)])


def _deg_call(dst, ew):
    return pl.kernel(
        _sc_deg_body,
        out_type=jax.ShapeDtypeStruct((NC, NP, 16), _f32),
        mesh=_mesh(),
        compiler_params=_sc_params,
        scratch_types=[
            pltpu.VMEM((CHD,), _i32),
            pltpu.VMEM((CHD,), _f32),
            pltpu.VMEM((CHD, 16), _f32),
            pltpu.VMEM_SHARED((NP, 16), _f32),
        ],
    )(dst, ew)


def _sc_scatter_body(hp_hbm, src_hbm, dst_hbm, ew_hbm, out_hbm,
                     src0, src1, dst0, dst1, ew0, ew1, rows0, rows1,
                     agg_sh, gsem, ssem, isem):
    srcs = (src0, src1)
    dsts = (dst0, dst1)
    ews = (ew0, ew1)
    rows = (rows0, rows1)
    c = lax.axis_index("c")
    s = lax.axis_index("s")
    wid = s * NC + c

    def zrow(r, carry):
        for j in range(8):
            rows0[r, pl.ds(j * 16, 16)] = jnp.zeros((16,), _f32)
        return carry

    lax.fori_loop(0, CH, zrow, 0)
    for t in range(RPT // CH):
        pltpu.sync_copy(rows0, agg_sh.at[pl.ds(s * RPT + t * CH, CH)])
    plsc.subcore_barrier()

    def issue_idx(i, p):
        b = (wid + NW * i) * CH
        pltpu.async_copy(src_hbm.at[pl.ds(b, CH)], srcs[p], isem)
        pltpu.async_copy(dst_hbm.at[pl.ds(b, CH)], dsts[p], isem)
        pltpu.async_copy(ew_hbm.at[pl.ds(b, CH)], ews[p], isem)

    def wait_idx(p):
        pltpu.make_async_copy(src_hbm.at[pl.ds(0, CH)], srcs[p], isem).wait()
        pltpu.make_async_copy(dst_hbm.at[pl.ds(0, CH)], dsts[p], isem).wait()
        pltpu.make_async_copy(ew_hbm.at[pl.ds(0, CH)], ews[p], isem).wait()

    def issue_gather(p):
        pltpu.async_copy(hp_hbm.at[srcs[p]], rows[p], gsem)

    def wait_gather(p):
        pltpu.make_async_copy(hp_hbm.at[srcs[p]], rows[p], gsem).wait()

    def issue_scatter(p):
        pltpu.async_copy(rows[p], agg_sh.at[dsts[p]], ssem, add=True)

    def wait_scatter(p):
        pltpu.make_async_copy(rows[p], agg_sh.at[dsts[p]], ssem).wait()

    def scale(p):
        ev = ews[p]
        rw = rows[p]

        def grp(g, cc):
            ew16 = ev[pl.ds(g * 16, 16)]
            for l in range(16):
                e = g * 16 + l
                sp = ew16[l]
                for j in range(8):
                    sl = pl.ds(j * 16, 16)
                    rw[e, sl] = rw[e, sl] * sp
            return cc

        lax.fori_loop(0, CH // 16, grp, 0)

    def half(i, p, drain, issue):
        if drain:
            wait_gather(p)
            wait_scatter(1 - p)
        else:
            wait_gather(p)
        if issue:
            issue_idx(i + 1, 1 - p)
        scale(p)
        issue_scatter(p)
        if issue:
            wait_idx(1 - p)
            issue_gather(1 - p)

    issue_idx(0, 0)
    wait_idx(0)
    issue_gather(0)

    half(0, 0, False, True)
    half(1, 1, True, True)

    def pair(k, carry):
        half(2 * k, 0, True, True)
        half(2 * k + 1, 1, True, True)
        return carry

    lax.fori_loop(1, CPW // 2 - 1, pair, 0)
    half(CPW - 2, 0, True, True)
    half(CPW - 1, 1, True, False)
    wait_scatter(1)

    plsc.subcore_barrier()
    for t in range(RPT // CH):
        r0 = s * RPT + t * CH
        pltpu.sync_copy(agg_sh.at[pl.ds(r0, CH)],
                        out_hbm.at[c, pl.ds(r0, CH)])


def _scatter_call(hp, src, dst, ew):
    return pl.kernel(
        _sc_scatter_body,
        out_type=jax.ShapeDtypeStruct((NC, NP, D), _f32),
        mesh=_mesh(),
        scratch_types=[
            pltpu.VMEM((CH,), _i32),
            pltpu.VMEM((CH,), _i32),
            pltpu.VMEM((CH,), _i32),
            pltpu.VMEM((CH,), _i32),
            pltpu.VMEM((CH,), _f32),
            pltpu.VMEM((CH,), _f32),
            pltpu.VMEM((CH, D), _f32),
            pltpu.VMEM((CH, D), _f32),
            pltpu.VMEM_SHARED((NP, D), _f32),
            pltpu.SemaphoreType.DMA,
            pltpu.SemaphoreType.DMA,
            pltpu.SemaphoreType.DMA,
        ],
    )(hp, src, dst, ew)


def _tc_pre_body(degp_ref, x_ref, w_ref, b_ref, hp_ref, dis_ref):
    a = degp_ref[...]
    deg = a[0, :N, 0:1] + a[1, :N, 0:1]
    dis = jnp.where(deg > 0, lax.rsqrt(deg), 0.0)
    h = jnp.dot(x_ref[...], w_ref[...], preferred_element_type=_f32)
    hp_ref[...] = (h + b_ref[...]) * dis
    dis_ref[...] = dis


def _tc_pre_call(degp, x, w, b):
    return pl.pallas_call(
        _tc_pre_body,
        out_shape=[jax.ShapeDtypeStruct((N, D), _f32),
                   jax.ShapeDtypeStruct((N, 1), _f32)],
    )(degp, x, w, b)


def _tc_mid_body(aggp_ref, dis_ref, g_ref, be_ref, w_ref, b_ref, hp_ref):
    a = aggp_ref[...]
    dis = dis_ref[...]
    out = (a[0, :N] + a[1, :N]) * dis
    mu = jnp.mean(out, axis=0, keepdims=True)
    xc = out - mu
    var = jnp.mean(xc * xc, axis=0, keepdims=True)
    y = xc * (g_ref[...] / jnp.sqrt(var + 1e-5)) + be_ref[...]
    y = jnp.maximum(y, 0.0)
    h = jnp.dot(y, w_ref[...], preferred_element_type=_f32)
    hp_ref[...] = (h + b_ref[...]) * dis


def _tc_mid_call(aggp, dis, g, be, w, b):
    return pl.pallas_call(
        _tc_mid_body,
        out_shape=jax.ShapeDtypeStruct((N, D), _f32),
    )(aggp, dis, g, be, w, b)


def _tc_post_body(aggp_ref, dis_ref, g_ref, be_ref, y_ref):
    a = aggp_ref[...]
    out = (a[0, :N] + a[1, :N]) * dis_ref[...]
    mu = jnp.mean(out, axis=0, keepdims=True)
    xc = out - mu
    var = jnp.mean(xc * xc, axis=0, keepdims=True)
    y_ref[...] = xc * (g_ref[...] / jnp.sqrt(var + 1e-5)) + be_ref[...]


def _tc_post_call(aggp, dis, g, be):
    return pl.pallas_call(
        _tc_post_body,
        out_shape=jax.ShapeDtypeStruct((N, D), _f32),
    )(aggp, dis, g, be)


def kernel(x, edge_index, edge_attr, W0, b0, gamma0, beta0, W1, b1, gamma1, beta1):
    pad = EP - E
    src = jnp.concatenate([edge_index[0].astype(_i32),
                           jnp.zeros((pad,), _i32)])
    dst = jnp.concatenate([edge_index[1].astype(_i32),
                           jnp.zeros((pad,), _i32)])
    ew = jnp.concatenate([edge_attr[:, 0], jnp.zeros((pad,), _f32)])

    degp = _deg_call(dst, ew)
    hp0, dis = _tc_pre_call(degp, x, W0, b0.reshape(1, D))
    agg0 = _scatter_call(hp0, src, dst, ew)
    hp1 = _tc_mid_call(agg0, dis, gamma0.reshape(1, D), beta0.reshape(1, D),
                       W1, b1.reshape(1, D))
    agg1 = _scatter_call(hp1, src, dst, ew)
    return _tc_post_call(agg1, dis, gamma1.reshape(1, D), beta1.reshape(1, D))
